# baseline (speedup 1.0000x reference)
import jax
import jax.numpy as jnp
from jax import lax
from jax.experimental import pallas as pl
from jax.experimental.pallas import tpu as pltpu
from jax.experimental.pallas import tpu_sc as plsc

NUM_EMB = 4
EMB_DIM = 16
RANK = 8
L = 16
NC, NS = 2, 16
NW = NC * NS

CHUNK = 2048
NBUF = 2


def _body(idx_hbm, u_hbm, v_hbm, out_hbm, idx_bufs, out_bufs, uv, vv, tab,
          isems, osems):
    n_rows = idx_hbm.shape[0]
    per_w = n_rows // NW
    wid = lax.axis_index("s") * NC + lax.axis_index("c")
    base = wid * per_w

    pltpu.sync_copy(u_hbm, uv)
    pltpu.sync_copy(v_hbm, vv)
    lanes = lax.iota(jnp.int32, L)
    u_vecs = [uv[pl.ds(0, L)], uv[pl.ds(L, L)]]
    for e in range(NUM_EMB):
        acc = jnp.zeros((L,), jnp.float32)
        for r in range(RANK):
            flat = e * RANK + r
            acc = acc + u_vecs[flat // L][flat % L] * vv[r, :]
        plsc.store_scatter(tab, [lanes * NUM_EMB + e], acc)

    n_chunks = per_w // CHUNK
    n_pairs = n_chunks // NBUF
    groups = CHUNK // L

    dbase = [jnp.full((L,), d * NUM_EMB, jnp.int32) for d in range(EMB_DIM)]
    lanes16 = [lanes * EMB_DIM + d for d in range(EMB_DIM)]

    def idx_copy(c, b):
        return pltpu.make_async_copy(
            idx_hbm.at[pl.ds(base + c * CHUNK, CHUNK)], idx_bufs[b], isems[b]
        )

    def out_copy(c, b):
        return pltpu.make_async_copy(
            out_bufs[b],
            out_hbm.at[pl.ds((base + c * CHUNK) * EMB_DIM, CHUNK * EMB_DIM)],
            osems[b],
        )

    for b in range(NBUF):
        idx_copy(b, b).start()

    def pair_body(p, _):
        for b in range(NBUF):
            c = p * NBUF + b
            idx_copy(c, b).wait()

            @pl.when(p > 0)
            def _():
                out_copy(c - NBUF, b).wait()

            @plsc.parallel_loop(0, groups, step=1, unroll=8)
            def group_body(g):
                idx_v = idx_bufs[b][pl.ds(g * L, L)]
                gbase = g * (L * EMB_DIM)
                for d in range(EMB_DIM):
                    col = plsc.load_gather(tab, [dbase[d] + idx_v])
                    plsc.store_scatter(out_bufs[b], [gbase + lanes16[d]], col)

            @pl.when(p + 1 < n_pairs)
            def _():
                idx_copy(c + NBUF, b).start()

            out_copy(c, b).start()
        return 0

    lax.fori_loop(0, n_pairs, pair_body, 0)
    for b in range(NBUF):
        out_copy(n_chunks - NBUF + b, b).wait()


def kernel(idx, U, V):
    B, Lseq = idx.shape
    n = B * Lseq
    idx_flat = idx.reshape(n).astype(jnp.int32)

    mesh = plsc.VectorSubcoreMesh(
        core_axis_name="c", subcore_axis_name="s", num_cores=NC, num_subcores=NS
    )
    run = pl.kernel(
        _body,
        out_type=jax.ShapeDtypeStruct((n * EMB_DIM,), jnp.float32),
        mesh=mesh,
        compiler_params=pltpu.CompilerParams(needs_layout_passes=False),
        scratch_types=[
            [pltpu.VMEM((CHUNK,), jnp.int32) for _ in range(NBUF)],
            [pltpu.VMEM((CHUNK * EMB_DIM,), jnp.float32) for _ in range(NBUF)],
            pltpu.VMEM((NUM_EMB * RANK,), jnp.float32),
            pltpu.VMEM((RANK, EMB_DIM), jnp.float32),
            pltpu.VMEM((NUM_EMB * EMB_DIM,), jnp.float32),
            [pltpu.SemaphoreType.DMA for _ in range(NBUF)],
            [pltpu.SemaphoreType.DMA for _ in range(NBUF)],
        ],
    )
    out = run(idx_flat, U.reshape(NUM_EMB * RANK), V)
    return out.reshape(B, Lseq, EMB_DIM)


# speedup vs baseline: 1.0806x; 1.0806x over previous
import jax
import jax.numpy as jnp
from jax import lax
from jax.experimental import pallas as pl
from jax.experimental.pallas import tpu as pltpu
from jax.experimental.pallas import tpu_sc as plsc

NUM_EMB = 4
EMB_DIM = 16
RANK = 8
L = 16
NC, NS = 2, 16
NW = NC * NS

CHUNK = 2048
NBUF = 2


def _body(idx_hbm, u_hbm, v_hbm, out_hbm, idx_bufs, out_bufs, uv, vv, tab,
          isems, osems):
    n_rows = idx_hbm.shape[0]
    per_w = n_rows // NW
    wid = lax.axis_index("s") * NC + lax.axis_index("c")
    base = wid * per_w

    pltpu.sync_copy(u_hbm, uv)
    pltpu.sync_copy(v_hbm, vv)
    lanes = lax.iota(jnp.int32, L)
    u_vecs = [uv[pl.ds(0, L)], uv[pl.ds(L, L)]]
    for e in range(NUM_EMB):
        acc = jnp.zeros((L,), jnp.float32)
        for r in range(RANK):
            flat = e * RANK + r
            acc = acc + u_vecs[flat // L][flat % L] * vv[r, :]
        plsc.store_scatter(tab, [lanes * NUM_EMB + e], acc)

    n_chunks = per_w // CHUNK
    n_pairs = n_chunks // NBUF
    groups = CHUNK // L

    dbase = [jnp.full((L,), d * NUM_EMB, jnp.int32) for d in range(EMB_DIM)]
    lanes16 = [lanes * EMB_DIM + d for d in range(EMB_DIM)]

    def idx_copy(c, b):
        return pltpu.make_async_copy(
            idx_hbm.at[pl.ds(base + c * CHUNK, CHUNK)], idx_bufs[b], isems[b]
        )

    def out_copy(c, b):
        return pltpu.make_async_copy(
            out_bufs[b],
            out_hbm.at[pl.ds((base + c * CHUNK) * EMB_DIM, CHUNK * EMB_DIM)],
            osems[b],
        )

    for b in range(NBUF):
        idx_copy(b, b).start()

    def pair_body(p, _):
        for b in range(NBUF):
            c = p * NBUF + b
            idx_copy(c, b).wait()

            @pl.when(p > 0)
            def _():
                out_copy(c - NBUF, b).wait()

            @plsc.parallel_loop(0, groups, step=1, unroll=2)
            def group_body(g):
                idx_v = idx_bufs[b][pl.ds(g * L, L)]
                gbase = g * (L * EMB_DIM)
                for d in range(EMB_DIM):
                    col = plsc.load_gather(tab, [dbase[d] + idx_v])
                    plsc.store_scatter(out_bufs[b], [gbase + lanes16[d]], col)

            @pl.when(p + 1 < n_pairs)
            def _():
                idx_copy(c + NBUF, b).start()

            out_copy(c, b).start()
        return 0

    lax.fori_loop(0, n_pairs, pair_body, 0)
    for b in range(NBUF):
        out_copy(n_chunks - NBUF + b, b).wait()


def kernel(idx, U, V):
    B, Lseq = idx.shape
    n = B * Lseq
    idx_flat = idx.reshape(n).astype(jnp.int32)

    mesh = plsc.VectorSubcoreMesh(
        core_axis_name="c", subcore_axis_name="s", num_cores=NC, num_subcores=NS
    )
    run = pl.kernel(
        _body,
        out_type=jax.ShapeDtypeStruct((n * EMB_DIM,), jnp.float32),
        mesh=mesh,
        compiler_params=pltpu.CompilerParams(needs_layout_passes=False),
        scratch_types=[
            [pltpu.VMEM((CHUNK,), jnp.int32) for _ in range(NBUF)],
            [pltpu.VMEM((CHUNK * EMB_DIM,), jnp.float32) for _ in range(NBUF)],
            pltpu.VMEM((NUM_EMB * RANK,), jnp.float32),
            pltpu.VMEM((RANK, EMB_DIM), jnp.float32),
            pltpu.VMEM((NUM_EMB * EMB_DIM,), jnp.float32),
            [pltpu.SemaphoreType.DMA for _ in range(NBUF)],
            [pltpu.SemaphoreType.DMA for _ in range(NBUF)],
        ],
    )
    out = run(idx_flat, U.reshape(NUM_EMB * RANK), V)
    return out.reshape(B, Lseq, EMB_DIM)


# speedup vs baseline: 1.0832x; 1.0024x over previous
import jax
import jax.numpy as jnp
from jax import lax
from jax.experimental import pallas as pl
from jax.experimental.pallas import tpu as pltpu
from jax.experimental.pallas import tpu_sc as plsc

NUM_EMB = 4
EMB_DIM = 16
RANK = 8
L = 16
NC, NS = 2, 16
NW = NC * NS

CHUNK = 2048
NBUF = 2


def _body(idx_hbm, u_hbm, v_hbm, out_hbm, idx_bufs, out_bufs, uv, vv, tab,
          isems, osems):
    n_rows = idx_hbm.shape[0]
    per_w = n_rows // NW
    wid = lax.axis_index("s") * NC + lax.axis_index("c")
    base = wid * per_w

    pltpu.sync_copy(u_hbm, uv)
    pltpu.sync_copy(v_hbm, vv)
    lanes = lax.iota(jnp.int32, L)
    u_vecs = [uv[pl.ds(0, L)], uv[pl.ds(L, L)]]
    for e in range(NUM_EMB):
        acc = jnp.zeros((L,), jnp.float32)
        for r in range(RANK):
            flat = e * RANK + r
            acc = acc + u_vecs[flat // L][flat % L] * vv[r, :]
        plsc.store_scatter(tab, [lanes * NUM_EMB + e], acc)

    n_chunks = per_w // CHUNK
    n_pairs = n_chunks // NBUF
    groups = CHUNK // L

    lanes16_base = lanes * EMB_DIM

    def idx_copy(c, b):
        return pltpu.make_async_copy(
            idx_hbm.at[pl.ds(base + c * CHUNK, CHUNK)], idx_bufs[b], isems[b]
        )

    def out_copy(c, b):
        return pltpu.make_async_copy(
            out_bufs[b],
            out_hbm.at[pl.ds((base + c * CHUNK) * EMB_DIM, CHUNK * EMB_DIM)],
            osems[b],
        )

    for b in range(NBUF):
        idx_copy(b, b).start()

    def pair_body(p, _):
        for b in range(NBUF):
            c = p * NBUF + b
            idx_copy(c, b).wait()

            @pl.when(p > 0)
            def _():
                out_copy(c - NBUF, b).wait()

            @plsc.parallel_loop(0, groups, step=1, unroll=4)
            def group_body(g):
                idx_v = idx_bufs[b][pl.ds(g * L, L)]
                gbase = g * (L * EMB_DIM)
                for d in range(EMB_DIM):
                    col = plsc.load_gather(tab, [idx_v + (d * NUM_EMB)])
                    plsc.store_scatter(
                        out_bufs[b], [lanes16_base + (gbase + d)], col
                    )

            @pl.when(p + 1 < n_pairs)
            def _():
                idx_copy(c + NBUF, b).start()

            out_copy(c, b).start()
        return 0

    lax.fori_loop(0, n_pairs, pair_body, 0)
    for b in range(NBUF):
        out_copy(n_chunks - NBUF + b, b).wait()


def kernel(idx, U, V):
    B, Lseq = idx.shape
    n = B * Lseq
    idx_flat = idx.reshape(n).astype(jnp.int32)

    mesh = plsc.VectorSubcoreMesh(
        core_axis_name="c", subcore_axis_name="s", num_cores=NC, num_subcores=NS
    )
    run = pl.kernel(
        _body,
        out_type=jax.ShapeDtypeStruct((n * EMB_DIM,), jnp.float32),
        mesh=mesh,
        compiler_params=pltpu.CompilerParams(needs_layout_passes=False),
        scratch_types=[
            [pltpu.VMEM((CHUNK,), jnp.int32) for _ in range(NBUF)],
            [pltpu.VMEM((CHUNK * EMB_DIM,), jnp.float32) for _ in range(NBUF)],
            pltpu.VMEM((NUM_EMB * RANK,), jnp.float32),
            pltpu.VMEM((RANK, EMB_DIM), jnp.float32),
            pltpu.VMEM((NUM_EMB * EMB_DIM,), jnp.float32),
            [pltpu.SemaphoreType.DMA for _ in range(NBUF)],
            [pltpu.SemaphoreType.DMA for _ in range(NBUF)],
        ],
    )
    out = run(idx_flat, U.reshape(NUM_EMB * RANK), V)
    return out.reshape(B, Lseq, EMB_DIM)


# speedup vs baseline: 1.1844x; 1.0934x over previous
import jax
import jax.numpy as jnp
from jax import lax
from jax.experimental import pallas as pl
from jax.experimental.pallas import tpu as pltpu
from jax.experimental.pallas import tpu_sc as plsc

NUM_EMB = 4
EMB_DIM = 16
RANK = 8
L = 16
NC, NS = 2, 16
NW = NC * NS

CHUNK = 2048
NBUF = 2


def _body(idx_hbm, u_hbm, v_hbm, out_hbm, idx_bufs, out_bufs, uv, vv, tab,
          isems, osems):
    n_rows = idx_hbm.shape[0]
    per_w = n_rows // NW
    wid = lax.axis_index("s") * NC + lax.axis_index("c")
    base = wid * per_w

    pltpu.sync_copy(u_hbm, uv)
    pltpu.sync_copy(v_hbm, vv)
    lanes = lax.iota(jnp.int32, L)
    u_vecs = [uv[pl.ds(0, L)], uv[pl.ds(L, L)]]
    for e in range(NUM_EMB):
        acc = jnp.zeros((L,), jnp.float32)
        for r in range(RANK):
            flat = e * RANK + r
            acc = acc + u_vecs[flat // L][flat % L] * vv[r, :]
        plsc.store_scatter(tab, [lanes * EMB_DIM + e], acc)

    n_chunks = per_w // CHUNK
    n_pairs = n_chunks // NBUF
    groups = CHUNK // L

    lanes16_base = lanes * EMB_DIM
    ecols = [tab[pl.ds(d * EMB_DIM, L)] for d in range(EMB_DIM)]

    def idx_copy(c, b):
        return pltpu.make_async_copy(
            idx_hbm.at[pl.ds(base + c * CHUNK, CHUNK)], idx_bufs[b], isems[b]
        )

    def out_copy(c, b):
        return pltpu.make_async_copy(
            out_bufs[b],
            out_hbm.at[pl.ds((base + c * CHUNK) * EMB_DIM, CHUNK * EMB_DIM)],
            osems[b],
        )

    for b in range(NBUF):
        idx_copy(b, b).start()

    def pair_body(p, _):
        for b in range(NBUF):
            c = p * NBUF + b
            idx_copy(c, b).wait()

            @pl.when(p > 0)
            def _():
                out_copy(c - NBUF, b).wait()

            @plsc.parallel_loop(0, groups, step=1, unroll=4)
            def group_body(g):
                idx_v = idx_bufs[b][pl.ds(g * L, L)]
                gbase = g * (L * EMB_DIM)
                for d in range(EMB_DIM):
                    col = lax.gather(
                        ecols[d],
                        idx_v[:, None],
                        dimension_numbers=lax.GatherDimensionNumbers(
                            offset_dims=(),
                            collapsed_slice_dims=(0,),
                            start_index_map=(0,),
                        ),
                        slice_sizes=(1,),
                        mode=lax.GatherScatterMode.PROMISE_IN_BOUNDS,
                    )
                    plsc.store_scatter(
                        out_bufs[b], [lanes16_base + (gbase + d)], col
                    )

            @pl.when(p + 1 < n_pairs)
            def _():
                idx_copy(c + NBUF, b).start()

            out_copy(c, b).start()
        return 0

    lax.fori_loop(0, n_pairs, pair_body, 0)
    for b in range(NBUF):
        out_copy(n_chunks - NBUF + b, b).wait()


def kernel(idx, U, V):
    B, Lseq = idx.shape
    n = B * Lseq
    idx_flat = idx.reshape(n).astype(jnp.int32)

    mesh = plsc.VectorSubcoreMesh(
        core_axis_name="c", subcore_axis_name="s", num_cores=NC, num_subcores=NS
    )
    run = pl.kernel(
        _body,
        out_type=jax.ShapeDtypeStruct((n * EMB_DIM,), jnp.float32),
        mesh=mesh,
        compiler_params=pltpu.CompilerParams(needs_layout_passes=False),
        scratch_types=[
            [pltpu.VMEM((CHUNK,), jnp.int32) for _ in range(NBUF)],
            [pltpu.VMEM((CHUNK * EMB_DIM,), jnp.float32) for _ in range(NBUF)],
            pltpu.VMEM((NUM_EMB * RANK,), jnp.float32),
            pltpu.VMEM((RANK, EMB_DIM), jnp.float32),
            pltpu.VMEM((EMB_DIM * EMB_DIM,), jnp.float32),
            [pltpu.SemaphoreType.DMA for _ in range(NBUF)],
            [pltpu.SemaphoreType.DMA for _ in range(NBUF)],
        ],
    )
    out = run(idx_flat, U.reshape(NUM_EMB * RANK), V)
    return out.reshape(B, Lseq, EMB_DIM)
